# Initial kernel scaffold; baseline (speedup 1.0000x reference)
import jax
import jax.numpy as jnp
from jax.experimental import pallas as pl


def kernel(h, edge_index, W1, b1, W2, b2, eps, bn_gamma, bn_beta):
    raise NotImplementedError("write your pallas kernel here")


# speedup vs baseline: 4.2558x; 4.2558x over previous
import functools

import jax
import jax.numpy as jnp
from jax import lax
from jax.experimental import pallas as pl
from jax.experimental.pallas import tpu as pltpu
from jax.experimental.pallas import tpu_sc as plsc

NC = 2
NS = 16
NW = NC * NS
CHUNK = 128


def _seg_sum_body(h_hbm, src_hbm, dst_hbm, zblk_hbm, out_hbm,
                  src_v, dst_v, rows_v, acc_sh, sem):
    cpt = src_v.shape[0]
    n_pad = acc_sh.shape[0]
    rows_per_tile = n_pad // NS
    cid = lax.axis_index("c")
    sid = lax.axis_index("s")
    wid = cid * NS + sid

    pltpu.sync_copy(src_hbm.at[wid], src_v)
    pltpu.sync_copy(dst_hbm.at[wid], dst_v)

    row0 = sid * rows_per_tile
    for r in range(rows_per_tile // CHUNK):
        pltpu.sync_copy(zblk_hbm, acc_sh.at[pl.ds(row0 + r * CHUNK, CHUNK)])
    plsc.subcore_barrier()

    def chunk_body(j, carry):
        pltpu.async_copy(h_hbm.at[src_v.at[j]], rows_v, sem).wait()
        pltpu.sync_copy(rows_v, acc_sh.at[dst_v.at[j]], add=True)
        return carry

    lax.fori_loop(0, cpt, chunk_body, 0)
    plsc.subcore_barrier()

    pltpu.sync_copy(acc_sh.at[pl.ds(row0, rows_per_tile)],
                    out_hbm.at[cid, pl.ds(row0, rows_per_tile)])


def _segment_sum_sc(h, src, dst, n_pad):
    n, d = h.shape
    e = src.shape[0]
    ept = -(-e // (NW * CHUNK)) * CHUNK
    e_pad = ept * NW
    cpt = ept // CHUNK
    src_p = jnp.concatenate(
        [src, jnp.zeros((e_pad - e,), jnp.int32)]).reshape(NW, cpt, CHUNK)
    dst_p = jnp.concatenate(
        [dst, jnp.full((e_pad - e,), n, jnp.int32)]).reshape(NW, cpt, CHUNK)
    zblk = jnp.zeros((CHUNK, d), jnp.float32)

    mesh = plsc.VectorSubcoreMesh(core_axis_name="c", subcore_axis_name="s",
                                  num_cores=NC, num_subcores=NS)
    grid_kernel = functools.partial(
        pl.kernel,
        out_type=jax.ShapeDtypeStruct((NC, n_pad, d), jnp.float32),
        mesh=mesh,
        scratch_types=[
            pltpu.VMEM((cpt, CHUNK), jnp.int32),
            pltpu.VMEM((cpt, CHUNK), jnp.int32),
            pltpu.VMEM((CHUNK, d), jnp.float32),
            pltpu.VMEM_SHARED((n_pad, d), jnp.float32),
            pltpu.SemaphoreType.DMA,
        ],
    )
    return grid_kernel(_seg_sum_body)(h, src_p, dst_p, zblk)


def _gin_tc_body(h_ref, parts_ref, w1_ref, b1_ref, w2_ref, b2_ref,
                 eps_ref, g_ref, bt_ref, out_ref):
    n = h_ref.shape[0]
    h = h_ref[...]
    neigh = parts_ref[0, :n, :] + parts_ref[1, :n, :]
    z = (1.0 + eps_ref[0, 0]) * h + neigh
    z = jnp.dot(z, w1_ref[...], preferred_element_type=jnp.float32)
    z = jnp.maximum(z + b1_ref[...], 0.0)
    z = jnp.dot(z, w2_ref[...], preferred_element_type=jnp.float32)
    z = z + b2_ref[...]
    mean = jnp.mean(z, axis=0, keepdims=True)
    var = jnp.mean((z - mean) ** 2, axis=0, keepdims=True)
    z = (z - mean) * lax.rsqrt(var + 1e-5) * g_ref[...] + bt_ref[...]
    out_ref[...] = h + jnp.maximum(z, 0.0)


def kernel(h, edge_index, W1, b1, W2, b2, eps, bn_gamma, bn_beta):
    n, d = h.shape
    n_pad = -(-(n + 1) // (NS * CHUNK)) * NS * CHUNK
    parts = _segment_sum_sc(h, edge_index[0], edge_index[1], n_pad)
    return pl.pallas_call(
        _gin_tc_body,
        out_shape=jax.ShapeDtypeStruct((n, d), jnp.float32),
    )(h, parts,
      W1, b1.reshape(1, d), W2, b2.reshape(1, d),
      jnp.reshape(eps, (1, 1)), bn_gamma.reshape(1, d), bn_beta.reshape(1, d))
